# Initial kernel scaffold; baseline (speedup 1.0000x reference)
import functools

import jax
import jax.numpy as jnp
from jax import lax
from jax.experimental import pallas as pl
from jax.experimental.pallas import tpu as pltpu
from jax.experimental.pallas import tpu_sc as plsc

_N = 10000
_E = 320000
_DIN = 128
_DH = 16
_DOUT = 10

_NC = 2
_NS = 16
_NP = 10240
_NPT = _NP // _NS
_EPW = _E // (_NC * _NS)
_C = 1000
_NCH = _EPW // _C

_mesh = plsc.VectorSubcoreMesh(
    core_axis_name="c", subcore_axis_name="s", num_cores=_NC, num_subcores=_NS
)


@functools.partial(
    pl.kernel,
    out_type=jax.ShapeDtypeStruct((_NC, _NP), jnp.float32),
    mesh=_mesh,
    scratch_types=[
        pltpu.VMEM((_C,), jnp.int32),
        pltpu.VMEM((_C,), jnp.float32),
        pltpu.VMEM((_NPT,), jnp.float32),
        pltpu.VMEM_SHARED((_NP,), jnp.float32),
    ],
)
def _sc_deg(dst_hbm, ew_hbm, out_hbm, idx_v, w_v, zer_v, deg_sh):
    cid = lax.axis_index("c")
    sid = lax.axis_index("s")

    def zbody(i, carry):
        zer_v[pl.ds(i * 16, 16)] = jnp.zeros((16,), jnp.float32)
        return carry

    lax.fori_loop(0, _NPT // 16, zbody, 0, unroll=8)
    pltpu.sync_copy(zer_v, deg_sh.at[pl.ds(sid * _NPT, _NPT)])
    plsc.subcore_barrier()

    base = cid * (_E // _NC) + sid * _EPW

    def chunk(k, carry):
        off = base + k * _C
        pltpu.sync_copy(dst_hbm.at[pl.ds(off, _C)], idx_v)
        pltpu.sync_copy(ew_hbm.at[pl.ds(off, _C)], w_v)
        pltpu.sync_copy(w_v, deg_sh.at[idx_v], add=True)
        return carry

    lax.fori_loop(0, _NCH, chunk, 0)
    plsc.subcore_barrier()
    pltpu.sync_copy(
        deg_sh.at[pl.ds(sid * _NPT, _NPT)],
        out_hbm.at[cid, pl.ds(sid * _NPT, _NPT)],
    )


@functools.partial(
    pl.kernel,
    out_type=jax.ShapeDtypeStruct((_NC, _NP, _DH), jnp.float32),
    mesh=_mesh,
    scratch_types=[
        pltpu.VMEM((_C,), jnp.int32),
        pltpu.VMEM((_C,), jnp.int32),
        pltpu.VMEM((_C,), jnp.float32),
        pltpu.VMEM((_C, _DH), jnp.float32),
        pltpu.VMEM((_NPT, _DH), jnp.float32),
        pltpu.VMEM_SHARED((_NP, _DH), jnp.float32),
        pltpu.SemaphoreType.DMA,
    ],
)
def _sc_agg(src_hbm, dst_hbm, ew_hbm, h_hbm, out_hbm,
            si_v, di_v, w_v, rows_v, zer_v, agg_sh, sem):
    cid = lax.axis_index("c")
    sid = lax.axis_index("s")

    def zbody(i, carry):
        zer_v[i] = jnp.zeros((_DH,), jnp.float32)
        return carry

    lax.fori_loop(0, _NPT, zbody, 0, unroll=8)
    pltpu.sync_copy(zer_v, agg_sh.at[pl.ds(sid * _NPT, _NPT)])
    plsc.subcore_barrier()

    base = cid * (_E // _NC) + sid * _EPW

    def chunk(k, carry):
        off = base + k * _C
        pltpu.sync_copy(src_hbm.at[pl.ds(off, _C)], si_v)
        gather = pltpu.async_copy(h_hbm.at[si_v], rows_v, sem)
        pltpu.sync_copy(ew_hbm.at[pl.ds(off, _C)], w_v)
        pltpu.sync_copy(dst_hbm.at[pl.ds(off, _C)], di_v)
        gather.wait()

        def ebody(j, c2):
            ws = plsc.load_gather(w_v, [jnp.full((16,), j, jnp.int32)])
            rows_v[j] = rows_v[j] * ws
            return c2

        lax.fori_loop(0, _C, ebody, 0, unroll=8)
        pltpu.sync_copy(rows_v, agg_sh.at[di_v], add=True)
        return carry

    lax.fori_loop(0, _NCH, chunk, 0)
    plsc.subcore_barrier()
    pltpu.sync_copy(
        agg_sh.at[pl.ds(sid * _NPT, _NPT)],
        out_hbm.at[cid, pl.ds(sid * _NPT, _NPT)],
    )


def _tc1_body(x_ref, w1_ref, degt_ref, h1s_ref, dis_ref):
    deg = degt_ref[:, 0:1] + degt_ref[:, 1:2] + 1.0
    dis = jnp.where(deg > 0, lax.rsqrt(jnp.maximum(deg, 1e-12)), 0.0)
    h1 = jnp.dot(x_ref[...], w1_ref[...], preferred_element_type=jnp.float32)
    h1s_ref[...] = h1 * dis
    dis_ref[...] = dis


_tc1 = pl.pallas_call(
    _tc1_body,
    out_shape=(
        jax.ShapeDtypeStruct((_NP, _DH), jnp.float32),
        jax.ShapeDtypeStruct((_NP, 1), jnp.float32),
    ),
)


def _tc2_body(aggp_ref, h1s_ref, dis_ref, b1_ref, w2_ref, zs_ref):
    dis = dis_ref[...]
    a1 = (aggp_ref[0] + aggp_ref[1] + h1s_ref[...]) * dis + b1_ref[...]
    h2 = jnp.maximum(a1, 0.0)
    z = jnp.dot(h2, w2_ref[...], preferred_element_type=jnp.float32)
    zs_ref[...] = z * dis


_tc2 = pl.pallas_call(
    _tc2_body,
    out_shape=jax.ShapeDtypeStruct((_NP, _DH), jnp.float32),
)


def _tc3_body(qp_ref, zs_ref, dis_ref, b2_ref, out_ref):
    dis = dis_ref[...]
    logits = (qp_ref[0] + qp_ref[1] + zs_ref[...]) * dis + b2_ref[...]
    mask = lax.broadcasted_iota(jnp.int32, (1, _DH), 1) < _DOUT
    lm = jnp.where(mask, logits, jnp.float32(-3.4e38))
    m = jnp.max(lm, axis=1, keepdims=True)
    e = jnp.where(mask, jnp.exp(logits - m), 0.0)
    s = jnp.sum(e, axis=1, keepdims=True)
    out_ref[...] = logits - m - jnp.log(s)


_tc3 = pl.pallas_call(
    _tc3_body,
    out_shape=jax.ShapeDtypeStruct((_NP, _DH), jnp.float32),
)


def kernel(x, edge_index, edge_weight, W1, b1, W2, b2):
    src = edge_index[0]
    dst = edge_index[1]
    xp = jnp.pad(x, ((0, _NP - _N), (0, 0)))
    w2p = jnp.pad(W2, ((0, 0), (0, _DH - _DOUT)))
    b1r = b1.reshape(1, _DH)
    b2r = jnp.pad(b2, (0, _DH - _DOUT)).reshape(1, _DH)

    degp = _sc_deg(dst, edge_weight)
    degt = degp.T
    h1s, dis = _tc1(xp, W1, degt)
    aggp = _sc_agg(src, dst, edge_weight, h1s)
    zs = _tc2(aggp, h1s, dis, b1r, w2p)
    qp = _sc_agg(src, dst, edge_weight, zs)
    outp = _tc3(qp, zs, dis, b2r)
    return outp[:_N, :_DOUT]


# speedup vs baseline: 34.6529x; 34.6529x over previous
import functools

import jax
import jax.numpy as jnp
from jax import lax
from jax.experimental import pallas as pl
from jax.experimental.pallas import tpu as pltpu
from jax.experimental.pallas import tpu_sc as plsc

_N = 10000
_E = 320000
_DIN = 128
_DH = 16
_DOUT = 10

_NC = 2
_NS = 16
_NP = 10240
_NPT = _NP // _NS
_EPW = _E // (_NC * _NS)
_C = 1000
_NCH = _EPW // _C

_mesh = plsc.VectorSubcoreMesh(
    core_axis_name="c", subcore_axis_name="s", num_cores=_NC, num_subcores=_NS
)


@functools.partial(
    pl.kernel,
    out_type=jax.ShapeDtypeStruct((_NC, _NP), jnp.float32),
    mesh=_mesh,
    scratch_types=[
        pltpu.VMEM((_C,), jnp.int32),
        pltpu.VMEM((_C,), jnp.float32),
        pltpu.VMEM((_NPT,), jnp.float32),
        pltpu.VMEM_SHARED((_NP,), jnp.float32),
    ],
    compiler_params=pltpu.CompilerParams(needs_layout_passes=False, use_tc_tiling_on_sc=False),
)
def _sc_deg(dst_hbm, ew_hbm, out_hbm, idx_v, w_v, zer_v, deg_sh):
    cid = lax.axis_index("c")
    sid = lax.axis_index("s")

    def zbody(i, carry):
        zer_v[pl.ds(i * 16, 16)] = jnp.zeros((16,), jnp.float32)
        return carry

    lax.fori_loop(0, _NPT // 16, zbody, 0, unroll=8)
    pltpu.sync_copy(zer_v, deg_sh.at[pl.ds(sid * _NPT, _NPT)])
    plsc.subcore_barrier()

    base = cid * (_E // _NC) + sid * _EPW

    def chunk(k, carry):
        off = base + k * _C
        pltpu.sync_copy(dst_hbm.at[pl.ds(off, _C)], idx_v)
        pltpu.sync_copy(ew_hbm.at[pl.ds(off, _C)], w_v)
        pltpu.sync_copy(w_v, deg_sh.at[idx_v], add=True)
        return carry

    lax.fori_loop(0, _NCH, chunk, 0)
    plsc.subcore_barrier()
    pltpu.sync_copy(
        deg_sh.at[pl.ds(sid * _NPT, _NPT)],
        out_hbm.at[cid, pl.ds(sid * _NPT, _NPT)],
    )


@functools.partial(
    pl.kernel,
    out_type=jax.ShapeDtypeStruct((_NC, _NP, _DH), jnp.float32),
    mesh=_mesh,
    scratch_types=[
        pltpu.VMEM((_C,), jnp.int32),
        pltpu.VMEM((_C,), jnp.int32),
        pltpu.VMEM((_C,), jnp.float32),
        pltpu.VMEM((_C, _DH), jnp.float32),
        pltpu.VMEM((_NPT, _DH), jnp.float32),
        pltpu.VMEM_SHARED((_NP, _DH), jnp.float32),
        pltpu.SemaphoreType.DMA,
    ],
    compiler_params=pltpu.CompilerParams(needs_layout_passes=False, use_tc_tiling_on_sc=False),
)
def _sc_agg(src_hbm, dst_hbm, ew_hbm, h_hbm, out_hbm,
            si_v, di_v, w_v, rows_v, zer_v, agg_sh, sem):
    cid = lax.axis_index("c")
    sid = lax.axis_index("s")

    def zbody(i, carry):
        zer_v[i] = jnp.zeros((_DH,), jnp.float32)
        return carry

    lax.fori_loop(0, _NPT, zbody, 0, unroll=8)
    pltpu.sync_copy(zer_v, agg_sh.at[pl.ds(sid * _NPT, _NPT)])
    plsc.subcore_barrier()

    base = cid * (_E // _NC) + sid * _EPW

    def chunk(k, carry):
        off = base + k * _C
        pltpu.sync_copy(src_hbm.at[pl.ds(off, _C)], si_v)
        gather = pltpu.async_copy(h_hbm.at[si_v], rows_v, sem)
        pltpu.sync_copy(ew_hbm.at[pl.ds(off, _C)], w_v)
        pltpu.sync_copy(dst_hbm.at[pl.ds(off, _C)], di_v)
        gather.wait()

        def ebody(j, c2):
            ws = plsc.load_gather(w_v, [jnp.full((16,), j, jnp.int32)])
            rows_v[j] = rows_v[j] * ws
            return c2

        lax.fori_loop(0, _C, ebody, 0, unroll=8)
        pltpu.sync_copy(rows_v, agg_sh.at[di_v], add=True)
        return carry

    lax.fori_loop(0, _NCH, chunk, 0)
    plsc.subcore_barrier()
    pltpu.sync_copy(
        agg_sh.at[pl.ds(sid * _NPT, _NPT)],
        out_hbm.at[cid, pl.ds(sid * _NPT, _NPT)],
    )


def _tc1_body(x_ref, w1_ref, degt_ref, h1s_ref, dis_ref):
    deg = degt_ref[:, 0:1] + degt_ref[:, 1:2] + 1.0
    dis = jnp.where(deg > 0, lax.rsqrt(jnp.maximum(deg, 1e-12)), 0.0)
    h1 = jnp.dot(x_ref[...], w1_ref[...], preferred_element_type=jnp.float32)
    h1s_ref[...] = h1 * dis
    dis_ref[...] = dis


_tc1 = pl.pallas_call(
    _tc1_body,
    out_shape=(
        jax.ShapeDtypeStruct((_NP, _DH), jnp.float32),
        jax.ShapeDtypeStruct((_NP, 1), jnp.float32),
    ),
)


def _tc2_body(aggp_ref, h1s_ref, dis_ref, b1_ref, w2_ref, zs_ref):
    dis = dis_ref[...]
    a1 = (aggp_ref[0] + aggp_ref[1] + h1s_ref[...]) * dis + b1_ref[...]
    h2 = jnp.maximum(a1, 0.0)
    z = jnp.dot(h2, w2_ref[...], preferred_element_type=jnp.float32)
    zs_ref[...] = z * dis


_tc2 = pl.pallas_call(
    _tc2_body,
    out_shape=jax.ShapeDtypeStruct((_NP, _DH), jnp.float32),
)


def _tc3_body(qp_ref, zs_ref, dis_ref, b2_ref, out_ref):
    dis = dis_ref[...]
    logits = (qp_ref[0] + qp_ref[1] + zs_ref[...]) * dis + b2_ref[...]
    mask = lax.broadcasted_iota(jnp.int32, (1, _DH), 1) < _DOUT
    lm = jnp.where(mask, logits, jnp.float32(-3.4e38))
    m = jnp.max(lm, axis=1, keepdims=True)
    e = jnp.where(mask, jnp.exp(logits - m), 0.0)
    s = jnp.sum(e, axis=1, keepdims=True)
    out_ref[...] = logits - m - jnp.log(s)


_tc3 = pl.pallas_call(
    _tc3_body,
    out_shape=jax.ShapeDtypeStruct((_NP, _DH), jnp.float32),
)


def kernel(x, edge_index, edge_weight, W1, b1, W2, b2):
    src = edge_index[0]
    dst = edge_index[1]
    xp = jnp.pad(x, ((0, _NP - _N), (0, 0)))
    w2p = jnp.pad(W2, ((0, 0), (0, _DH - _DOUT)))
    b1r = b1.reshape(1, _DH)
    b2r = jnp.pad(b2, (0, _DH - _DOUT)).reshape(1, _DH)

    degp = _sc_deg(dst, edge_weight)
    degt = degp.T
    h1s, dis = _tc1(xp, W1, degt)
    aggp = _sc_agg(src, dst, edge_weight, h1s)
    zs = _tc2(aggp, h1s, dis, b1r, w2p)
    qp = _sc_agg(src, dst, edge_weight, zs)
    outp = _tc3(qp, zs, dis, b2r)
    return outp[:_N, :_DOUT]


# speedup vs baseline: 48.1136x; 1.3884x over previous
import functools

import jax
import jax.numpy as jnp
from jax import lax
from jax.experimental import pallas as pl
from jax.experimental.pallas import tpu as pltpu
from jax.experimental.pallas import tpu_sc as plsc

_N = 10000
_E = 320000
_DIN = 128
_DH = 16
_DOUT = 10

_NC = 2
_NS = 16
_NP = 10240
_NPT = _NP // _NS
_EPW = _E // (_NC * _NS)
_C = 1000
_NCH = _EPW // _C

_mesh = plsc.VectorSubcoreMesh(
    core_axis_name="c", subcore_axis_name="s", num_cores=_NC, num_subcores=_NS
)


@functools.partial(
    pl.kernel,
    out_type=jax.ShapeDtypeStruct((_NC, _NP), jnp.float32),
    mesh=_mesh,
    scratch_types=[
        pltpu.VMEM((_C,), jnp.int32),
        pltpu.VMEM((_C,), jnp.float32),
        pltpu.VMEM((_NPT,), jnp.float32),
        pltpu.VMEM_SHARED((_NP,), jnp.float32),
    ],
    compiler_params=pltpu.CompilerParams(needs_layout_passes=False, use_tc_tiling_on_sc=False),
)
def _sc_deg(dst_hbm, ew_hbm, out_hbm, idx_v, w_v, zer_v, deg_sh):
    cid = lax.axis_index("c")
    sid = lax.axis_index("s")

    def zbody(i, carry):
        zer_v[pl.ds(i * 16, 16)] = jnp.zeros((16,), jnp.float32)
        return carry

    lax.fori_loop(0, _NPT // 16, zbody, 0, unroll=8)
    pltpu.sync_copy(zer_v, deg_sh.at[pl.ds(sid * _NPT, _NPT)])
    plsc.subcore_barrier()

    base = cid * (_E // _NC) + sid * _EPW

    def chunk(k, carry):
        off = base + k * _C
        pltpu.sync_copy(dst_hbm.at[pl.ds(off, _C)], idx_v)
        pltpu.sync_copy(ew_hbm.at[pl.ds(off, _C)], w_v)
        pltpu.sync_copy(w_v, deg_sh.at[idx_v], add=True)
        return carry

    lax.fori_loop(0, _NCH, chunk, 0)
    plsc.subcore_barrier()
    pltpu.sync_copy(
        deg_sh.at[pl.ds(sid * _NPT, _NPT)],
        out_hbm.at[cid, pl.ds(sid * _NPT, _NPT)],
    )


@functools.partial(
    pl.kernel,
    out_type=jax.ShapeDtypeStruct((_NC, _NP, _DH), jnp.float32),
    mesh=_mesh,
    scratch_types=[
        pltpu.VMEM((_C,), jnp.int32),
        pltpu.VMEM((_C,), jnp.int32),
        pltpu.VMEM((_C,), jnp.float32),
        pltpu.VMEM((_C, _DH), jnp.float32),
        pltpu.VMEM((_NPT, _DH), jnp.float32),
        pltpu.VMEM_SHARED((_NP, _DH), jnp.float32),
        pltpu.SemaphoreType.DMA,
    ],
    compiler_params=pltpu.CompilerParams(needs_layout_passes=False, use_tc_tiling_on_sc=False),
)
def _sc_agg(src_hbm, dst_hbm, ew_hbm, h_hbm, out_hbm,
            si_v, di_v, w_v, rows_v, zer_v, agg_sh, sem):
    cid = lax.axis_index("c")
    sid = lax.axis_index("s")

    def zbody(i, carry):
        zer_v[i] = jnp.zeros((_DH,), jnp.float32)
        return carry

    lax.fori_loop(0, _NPT, zbody, 0, unroll=8)
    pltpu.sync_copy(zer_v, agg_sh.at[pl.ds(sid * _NPT, _NPT)])
    plsc.subcore_barrier()

    base = cid * (_E // _NC) + sid * _EPW

    def chunk(k, carry):
        off = base + k * _C
        pltpu.sync_copy(src_hbm.at[pl.ds(off, _C)], si_v)
        gather = pltpu.async_copy(h_hbm.at[si_v], rows_v, sem)
        pltpu.sync_copy(ew_hbm.at[pl.ds(off, _C)], w_v)
        pltpu.sync_copy(dst_hbm.at[pl.ds(off, _C)], di_v)
        gather.wait()

        @plsc.parallel_loop(0, _C, unroll=8)
        def ebody(j):
            ws = plsc.load_gather(w_v, [jnp.full((16,), j, jnp.int32)])
            rows_v[j] = rows_v[j] * ws
        pltpu.sync_copy(rows_v, agg_sh.at[di_v], add=True)
        return carry

    lax.fori_loop(0, _NCH, chunk, 0)
    plsc.subcore_barrier()
    pltpu.sync_copy(
        agg_sh.at[pl.ds(sid * _NPT, _NPT)],
        out_hbm.at[cid, pl.ds(sid * _NPT, _NPT)],
    )


def _tc1_body(x_ref, w1_ref, degt_ref, h1s_ref, dis_ref):
    deg = degt_ref[:, 0:1] + degt_ref[:, 1:2] + 1.0
    dis = jnp.where(deg > 0, lax.rsqrt(jnp.maximum(deg, 1e-12)), 0.0)
    h1 = jnp.dot(x_ref[...], w1_ref[...], preferred_element_type=jnp.float32)
    h1s_ref[...] = h1 * dis
    dis_ref[...] = dis


_tc1 = pl.pallas_call(
    _tc1_body,
    out_shape=(
        jax.ShapeDtypeStruct((_NP, _DH), jnp.float32),
        jax.ShapeDtypeStruct((_NP, 1), jnp.float32),
    ),
)


def _tc2_body(aggp_ref, h1s_ref, dis_ref, b1_ref, w2_ref, zs_ref):
    dis = dis_ref[...]
    a1 = (aggp_ref[0] + aggp_ref[1] + h1s_ref[...]) * dis + b1_ref[...]
    h2 = jnp.maximum(a1, 0.0)
    z = jnp.dot(h2, w2_ref[...], preferred_element_type=jnp.float32)
    zs_ref[...] = z * dis


_tc2 = pl.pallas_call(
    _tc2_body,
    out_shape=jax.ShapeDtypeStruct((_NP, _DH), jnp.float32),
)


def _tc3_body(qp_ref, zs_ref, dis_ref, b2_ref, out_ref):
    dis = dis_ref[...]
    logits = (qp_ref[0] + qp_ref[1] + zs_ref[...]) * dis + b2_ref[...]
    mask = lax.broadcasted_iota(jnp.int32, (1, _DH), 1) < _DOUT
    lm = jnp.where(mask, logits, jnp.float32(-3.4e38))
    m = jnp.max(lm, axis=1, keepdims=True)
    e = jnp.where(mask, jnp.exp(logits - m), 0.0)
    s = jnp.sum(e, axis=1, keepdims=True)
    out_ref[...] = logits - m - jnp.log(s)


_tc3 = pl.pallas_call(
    _tc3_body,
    out_shape=jax.ShapeDtypeStruct((_NP, _DH), jnp.float32),
)


def kernel(x, edge_index, edge_weight, W1, b1, W2, b2):
    src = edge_index[0]
    dst = edge_index[1]
    xp = jnp.pad(x, ((0, _NP - _N), (0, 0)))
    w2p = jnp.pad(W2, ((0, 0), (0, _DH - _DOUT)))
    b1r = b1.reshape(1, _DH)
    b2r = jnp.pad(b2, (0, _DH - _DOUT)).reshape(1, _DH)

    degp = _sc_deg(dst, edge_weight)
    degt = degp.T
    h1s, dis = _tc1(xp, W1, degt)
    aggp = _sc_agg(src, dst, edge_weight, h1s)
    zs = _tc2(aggp, h1s, dis, b1r, w2p)
    qp = _sc_agg(src, dst, edge_weight, zs)
    outp = _tc3(qp, zs, dis, b2r)
    return outp[:_N, :_DOUT]


# speedup vs baseline: 53.7193x; 1.1165x over previous
import functools

import jax
import jax.numpy as jnp
from jax import lax
from jax.experimental import pallas as pl
from jax.experimental.pallas import tpu as pltpu
from jax.experimental.pallas import tpu_sc as plsc

_N = 10000
_E = 320000
_DIN = 128
_DH = 16
_DOUT = 10

_NC = 2
_NS = 16
_NP = 10240
_NPT = _NP // _NS
_EPW = _E // (_NC * _NS)
_C = 1000
_NCH = _EPW // _C

_mesh = plsc.VectorSubcoreMesh(
    core_axis_name="c", subcore_axis_name="s", num_cores=_NC, num_subcores=_NS
)


@functools.partial(
    pl.kernel,
    out_type=jax.ShapeDtypeStruct((_NC, _NP), jnp.float32),
    mesh=_mesh,
    scratch_types=[
        pltpu.VMEM((_C,), jnp.int32),
        pltpu.VMEM((_C,), jnp.float32),
        pltpu.VMEM((_NPT,), jnp.float32),
        pltpu.VMEM_SHARED((_NP,), jnp.float32),
    ],
    compiler_params=pltpu.CompilerParams(needs_layout_passes=False, use_tc_tiling_on_sc=False),
)
def _sc_deg(dst_hbm, ew_hbm, out_hbm, idx_v, w_v, zer_v, deg_sh):
    cid = lax.axis_index("c")
    sid = lax.axis_index("s")

    def zbody(i, carry):
        zer_v[pl.ds(i * 16, 16)] = jnp.zeros((16,), jnp.float32)
        return carry

    lax.fori_loop(0, _NPT // 16, zbody, 0, unroll=8)
    pltpu.sync_copy(zer_v, deg_sh.at[pl.ds(sid * _NPT, _NPT)])
    plsc.subcore_barrier()

    base = cid * (_E // _NC) + sid * _EPW

    def chunk(k, carry):
        off = base + k * _C
        pltpu.sync_copy(dst_hbm.at[pl.ds(off, _C)], idx_v)
        pltpu.sync_copy(ew_hbm.at[pl.ds(off, _C)], w_v)
        pltpu.sync_copy(w_v, deg_sh.at[idx_v], add=True)
        return carry

    lax.fori_loop(0, _NCH, chunk, 0)
    plsc.subcore_barrier()
    pltpu.sync_copy(
        deg_sh.at[pl.ds(sid * _NPT, _NPT)],
        out_hbm.at[cid, pl.ds(sid * _NPT, _NPT)],
    )


_NB = 3


@functools.partial(
    pl.kernel,
    out_type=jax.ShapeDtypeStruct((_NC, _NP, _DH), jnp.float32),
    mesh=_mesh,
    scratch_types=[
        pltpu.VMEM((_NB, _C), jnp.int32),
        pltpu.VMEM((_NB, _C), jnp.int32),
        pltpu.VMEM((_NB, _C), jnp.float32),
        pltpu.VMEM((_NB, _C, _DH), jnp.float32),
        pltpu.VMEM((_NPT, _DH), jnp.float32),
        pltpu.VMEM_SHARED((_NP, _DH), jnp.float32),
        [pltpu.SemaphoreType.DMA] * _NB,
        [pltpu.SemaphoreType.DMA] * _NB,
    ],
    compiler_params=pltpu.CompilerParams(needs_layout_passes=False, use_tc_tiling_on_sc=False),
)
def _sc_agg(src_hbm, dst_hbm, ew_hbm, h_hbm, out_hbm,
            si_v, di_v, w_v, rows_v, zer_v, agg_sh, gsems, ssems):
    cid = lax.axis_index("c")
    sid = lax.axis_index("s")

    @plsc.parallel_loop(0, _NPT, unroll=8)
    def zbody(i):
        zer_v[i] = jnp.zeros((_DH,), jnp.float32)

    pltpu.sync_copy(zer_v, agg_sh.at[pl.ds(sid * _NPT, _NPT)])
    plsc.subcore_barrier()

    base = cid * (_E // _NC) + sid * _EPW

    gds = [None] * _NCH
    sds = [None] * _NCH

    def prefetch(k):
        b = k % _NB
        off = base + k * _C
        pltpu.sync_copy(src_hbm.at[pl.ds(off, _C)], si_v.at[b])
        pltpu.sync_copy(dst_hbm.at[pl.ds(off, _C)], di_v.at[b])
        pltpu.sync_copy(ew_hbm.at[pl.ds(off, _C)], w_v.at[b])
        gds[k] = pltpu.async_copy(h_hbm.at[si_v.at[b]], rows_v.at[b],
                                  gsems[b])

    prefetch(0)
    for k in range(_NCH):
        b = k % _NB
        if k - 2 >= 0:
            sds[k - 2].wait()
        if k + 1 < _NCH:
            prefetch(k + 1)
        gds[k].wait()

        wb = w_v.at[b]
        rb = rows_v.at[b]

        @plsc.parallel_loop(0, _C, unroll=8)
        def ebody(j):
            ws = plsc.load_gather(wb, [jnp.full((16,), j, jnp.int32)])
            rb[j] = rb[j] * ws

        sds[k] = pltpu.async_copy(rows_v.at[b], agg_sh.at[di_v.at[b]],
                                  ssems[b], add=True)
    for k in range(max(0, _NCH - 2), _NCH):
        sds[k].wait()
    plsc.subcore_barrier()
    pltpu.sync_copy(
        agg_sh.at[pl.ds(sid * _NPT, _NPT)],
        out_hbm.at[cid, pl.ds(sid * _NPT, _NPT)],
    )


def _tc1_body(x_ref, w1_ref, degt_ref, h1s_ref, dis_ref):
    deg = degt_ref[:, 0:1] + degt_ref[:, 1:2] + 1.0
    dis = jnp.where(deg > 0, lax.rsqrt(jnp.maximum(deg, 1e-12)), 0.0)
    h1 = jnp.dot(x_ref[...], w1_ref[...], preferred_element_type=jnp.float32)
    h1s_ref[...] = h1 * dis
    dis_ref[...] = dis


_tc1 = pl.pallas_call(
    _tc1_body,
    out_shape=(
        jax.ShapeDtypeStruct((_NP, _DH), jnp.float32),
        jax.ShapeDtypeStruct((_NP, 1), jnp.float32),
    ),
)


def _tc2_body(aggp_ref, h1s_ref, dis_ref, b1_ref, w2_ref, zs_ref):
    dis = dis_ref[...]
    a1 = (aggp_ref[0] + aggp_ref[1] + h1s_ref[...]) * dis + b1_ref[...]
    h2 = jnp.maximum(a1, 0.0)
    z = jnp.dot(h2, w2_ref[...], preferred_element_type=jnp.float32)
    zs_ref[...] = z * dis


_tc2 = pl.pallas_call(
    _tc2_body,
    out_shape=jax.ShapeDtypeStruct((_NP, _DH), jnp.float32),
)


def _tc3_body(qp_ref, zs_ref, dis_ref, b2_ref, out_ref):
    dis = dis_ref[...]
    logits = (qp_ref[0] + qp_ref[1] + zs_ref[...]) * dis + b2_ref[...]
    mask = lax.broadcasted_iota(jnp.int32, (1, _DH), 1) < _DOUT
    lm = jnp.where(mask, logits, jnp.float32(-3.4e38))
    m = jnp.max(lm, axis=1, keepdims=True)
    e = jnp.where(mask, jnp.exp(logits - m), 0.0)
    s = jnp.sum(e, axis=1, keepdims=True)
    out_ref[...] = logits - m - jnp.log(s)


_tc3 = pl.pallas_call(
    _tc3_body,
    out_shape=jax.ShapeDtypeStruct((_NP, _DH), jnp.float32),
)


def kernel(x, edge_index, edge_weight, W1, b1, W2, b2):
    src = edge_index[0]
    dst = edge_index[1]
    xp = jnp.pad(x, ((0, _NP - _N), (0, 0)))
    w2p = jnp.pad(W2, ((0, 0), (0, _DH - _DOUT)))
    b1r = b1.reshape(1, _DH)
    b2r = jnp.pad(b2, (0, _DH - _DOUT)).reshape(1, _DH)

    degp = _sc_deg(dst, edge_weight)
    degt = degp.T
    h1s, dis = _tc1(xp, W1, degt)
    aggp = _sc_agg(src, dst, edge_weight, h1s)
    zs = _tc2(aggp, h1s, dis, b1r, w2p)
    qp = _sc_agg(src, dst, edge_weight, zs)
    outp = _tc3(qp, zs, dis, b2r)
    return outp[:_N, :_DOUT]


# speedup vs baseline: 53.8150x; 1.0018x over previous
import functools

import jax
import jax.numpy as jnp
from jax import lax
from jax.experimental import pallas as pl
from jax.experimental.pallas import tpu as pltpu
from jax.experimental.pallas import tpu_sc as plsc

_N = 10000
_E = 320000
_DIN = 128
_DH = 16
_DOUT = 10

_NC = 2
_NS = 16
_NP = 10240
_NPT = _NP // _NS
_EPW = _E // (_NC * _NS)
_C = 1000
_NCH = _EPW // _C

_mesh = plsc.VectorSubcoreMesh(
    core_axis_name="c", subcore_axis_name="s", num_cores=_NC, num_subcores=_NS
)


@functools.partial(
    pl.kernel,
    out_type=jax.ShapeDtypeStruct((_NC, _NP), jnp.float32),
    mesh=_mesh,
    scratch_types=[
        pltpu.VMEM((3, _C), jnp.int32),
        pltpu.VMEM((3, _C), jnp.float32),
        pltpu.VMEM((_NPT,), jnp.float32),
        pltpu.VMEM_SHARED((_NP,), jnp.float32),
        [pltpu.SemaphoreType.DMA] * 3,
    ],
    compiler_params=pltpu.CompilerParams(needs_layout_passes=False, use_tc_tiling_on_sc=False),
)
def _sc_deg(dst_hbm, ew_hbm, out_hbm, idx_v, w_v, zer_v, deg_sh, ssems):
    cid = lax.axis_index("c")
    sid = lax.axis_index("s")

    @plsc.parallel_loop(0, _NPT // 16, unroll=8)
    def zbody(i):
        zer_v[pl.ds(i * 16, 16)] = jnp.zeros((16,), jnp.float32)

    pltpu.sync_copy(zer_v, deg_sh.at[pl.ds(sid * _NPT, _NPT)])
    plsc.subcore_barrier()

    base = cid * (_E // _NC) + sid * _EPW

    def load(k):
        b = k % 3
        off = base + k * _C
        pltpu.sync_copy(dst_hbm.at[pl.ds(off, _C)], idx_v.at[b])
        pltpu.sync_copy(ew_hbm.at[pl.ds(off, _C)], w_v.at[b])

    sds = [None] * _NCH
    load(0)
    for k in range(_NCH):
        b = k % 3
        if k - 2 >= 0:
            sds[k - 2].wait()
        if k + 1 < _NCH:
            load(k + 1)
        sds[k] = pltpu.async_copy(w_v.at[b], deg_sh.at[idx_v.at[b]],
                                  ssems[b], add=True)
    for k in range(max(0, _NCH - 2), _NCH):
        sds[k].wait()
    plsc.subcore_barrier()
    pltpu.sync_copy(
        deg_sh.at[pl.ds(sid * _NPT, _NPT)],
        out_hbm.at[cid, pl.ds(sid * _NPT, _NPT)],
    )


_NB = 3


@functools.partial(
    pl.kernel,
    out_type=jax.ShapeDtypeStruct((_NC, _NP, _DH), jnp.float32),
    mesh=_mesh,
    scratch_types=[
        pltpu.VMEM((_NB, _C), jnp.int32),
        pltpu.VMEM((_NB, _C), jnp.int32),
        pltpu.VMEM((_NB, _C), jnp.float32),
        pltpu.VMEM((_NB, _C, _DH), jnp.float32),
        pltpu.VMEM((_NPT, _DH), jnp.float32),
        pltpu.VMEM_SHARED((_NP, _DH), jnp.float32),
        [pltpu.SemaphoreType.DMA] * _NB,
        [pltpu.SemaphoreType.DMA] * _NB,
    ],
    compiler_params=pltpu.CompilerParams(needs_layout_passes=False, use_tc_tiling_on_sc=False),
)
def _sc_agg(src_hbm, dst_hbm, ew_hbm, h_hbm, out_hbm,
            si_v, di_v, w_v, rows_v, zer_v, agg_sh, gsems, ssems):
    cid = lax.axis_index("c")
    sid = lax.axis_index("s")

    @plsc.parallel_loop(0, _NPT, unroll=8)
    def zbody(i):
        zer_v[i] = jnp.zeros((_DH,), jnp.float32)

    pltpu.sync_copy(zer_v, agg_sh.at[pl.ds(sid * _NPT, _NPT)])
    plsc.subcore_barrier()

    base = cid * (_E // _NC) + sid * _EPW

    gds = [None] * _NCH
    sds = [None] * _NCH

    def prefetch(k):
        b = k % _NB
        off = base + k * _C
        pltpu.sync_copy(src_hbm.at[pl.ds(off, _C)], si_v.at[b])
        pltpu.sync_copy(dst_hbm.at[pl.ds(off, _C)], di_v.at[b])
        pltpu.sync_copy(ew_hbm.at[pl.ds(off, _C)], w_v.at[b])
        gds[k] = pltpu.async_copy(h_hbm.at[si_v.at[b]], rows_v.at[b],
                                  gsems[b])

    prefetch(0)
    for k in range(_NCH):
        b = k % _NB
        if k - 2 >= 0:
            sds[k - 2].wait()
        if k + 1 < _NCH:
            prefetch(k + 1)
        gds[k].wait()

        wb = w_v.at[b]
        rb = rows_v.at[b]

        @plsc.parallel_loop(0, _C, unroll=16)
        def ebody(j):
            ws = plsc.load_gather(wb, [jnp.full((16,), j, jnp.int32)])
            rb[j] = rb[j] * ws

        sds[k] = pltpu.async_copy(rows_v.at[b], agg_sh.at[di_v.at[b]],
                                  ssems[b], add=True)
    for k in range(max(0, _NCH - 2), _NCH):
        sds[k].wait()
    plsc.subcore_barrier()
    pltpu.sync_copy(
        agg_sh.at[pl.ds(sid * _NPT, _NPT)],
        out_hbm.at[cid, pl.ds(sid * _NPT, _NPT)],
    )


def _tc1_body(x_ref, w1_ref, degt_ref, h1s_ref, dis_ref):
    deg = degt_ref[:, 0:1] + degt_ref[:, 1:2] + 1.0
    dis = jnp.where(deg > 0, lax.rsqrt(jnp.maximum(deg, 1e-12)), 0.0)
    h1 = jnp.dot(x_ref[...], w1_ref[...], preferred_element_type=jnp.float32)
    h1s_ref[...] = h1 * dis
    dis_ref[...] = dis


_tc1 = pl.pallas_call(
    _tc1_body,
    out_shape=(
        jax.ShapeDtypeStruct((_NP, _DH), jnp.float32),
        jax.ShapeDtypeStruct((_NP, 1), jnp.float32),
    ),
)


def _tc2_body(aggp_ref, h1s_ref, dis_ref, b1_ref, w2_ref, zs_ref):
    dis = dis_ref[...]
    a1 = (aggp_ref[0] + aggp_ref[1] + h1s_ref[...]) * dis + b1_ref[...]
    h2 = jnp.maximum(a1, 0.0)
    z = jnp.dot(h2, w2_ref[...], preferred_element_type=jnp.float32)
    zs_ref[...] = z * dis


_tc2 = pl.pallas_call(
    _tc2_body,
    out_shape=jax.ShapeDtypeStruct((_NP, _DH), jnp.float32),
)


def _tc3_body(qp_ref, zs_ref, dis_ref, b2_ref, out_ref):
    dis = dis_ref[...]
    logits = (qp_ref[0] + qp_ref[1] + zs_ref[...]) * dis + b2_ref[...]
    mask = lax.broadcasted_iota(jnp.int32, (1, _DH), 1) < _DOUT
    lm = jnp.where(mask, logits, jnp.float32(-3.4e38))
    m = jnp.max(lm, axis=1, keepdims=True)
    e = jnp.where(mask, jnp.exp(logits - m), 0.0)
    s = jnp.sum(e, axis=1, keepdims=True)
    out_ref[...] = logits - m - jnp.log(s)


_tc3 = pl.pallas_call(
    _tc3_body,
    out_shape=jax.ShapeDtypeStruct((_NP, _DH), jnp.float32),
)


def kernel(x, edge_index, edge_weight, W1, b1, W2, b2):
    src = edge_index[0]
    dst = edge_index[1]
    xp = jnp.pad(x, ((0, _NP - _N), (0, 0)))
    w2p = jnp.pad(W2, ((0, 0), (0, _DH - _DOUT)))
    b1r = b1.reshape(1, _DH)
    b2r = jnp.pad(b2, (0, _DH - _DOUT)).reshape(1, _DH)

    degp = _sc_deg(dst, edge_weight)
    degt = degp.T
    h1s, dis = _tc1(xp, W1, degt)
    aggp = _sc_agg(src, dst, edge_weight, h1s)
    zs = _tc2(aggp, h1s, dis, b1r, w2p)
    qp = _sc_agg(src, dst, edge_weight, zs)
    outp = _tc3(qp, zs, dis, b2r)
    return outp[:_N, :_DOUT]


# speedup vs baseline: 56.6808x; 1.0533x over previous
import functools

import jax
import jax.numpy as jnp
from jax import lax
from jax.experimental import pallas as pl
from jax.experimental.pallas import tpu as pltpu
from jax.experimental.pallas import tpu_sc as plsc

_N = 10000
_E = 320000
_DIN = 128
_DH = 16
_DOUT = 10

_NC = 2
_NS = 16
_NP = 10240
_NPT = _NP // _NS
_EPW = _E // (_NC * _NS)
_C = 1000
_NCH = _EPW // _C

_mesh = plsc.VectorSubcoreMesh(
    core_axis_name="c", subcore_axis_name="s", num_cores=_NC, num_subcores=_NS
)


@functools.partial(
    pl.kernel,
    out_type=jax.ShapeDtypeStruct((_NC, _NP), jnp.float32),
    mesh=_mesh,
    scratch_types=[
        pltpu.VMEM((3, _C), jnp.int32),
        pltpu.VMEM((3, _C), jnp.float32),
        pltpu.VMEM((_NPT,), jnp.float32),
        pltpu.VMEM_SHARED((_NP,), jnp.float32),
        [pltpu.SemaphoreType.DMA] * 3,
    ],
    compiler_params=pltpu.CompilerParams(needs_layout_passes=False, use_tc_tiling_on_sc=False),
)
def _sc_deg(ei_hbm, ew_hbm, out_hbm, idx_v, w_v, zer_v, deg_sh, ssems):
    cid = lax.axis_index("c")
    sid = lax.axis_index("s")

    @plsc.parallel_loop(0, _NPT // 16, unroll=8)
    def zbody(i):
        zer_v[pl.ds(i * 16, 16)] = jnp.zeros((16,), jnp.float32)

    pltpu.sync_copy(zer_v, deg_sh.at[pl.ds(sid * _NPT, _NPT)])
    plsc.subcore_barrier()

    base = cid * (_E // _NC) + sid * _EPW

    def load(k):
        b = k % 3
        off = base + k * _C
        pltpu.sync_copy(ei_hbm.at[1, pl.ds(off, _C)], idx_v.at[b])
        pltpu.sync_copy(ew_hbm.at[pl.ds(off, _C)], w_v.at[b])

    sds = [None] * _NCH
    load(0)
    for k in range(_NCH):
        b = k % 3
        if k - 2 >= 0:
            sds[k - 2].wait()
        if k + 1 < _NCH:
            load(k + 1)
        sds[k] = pltpu.async_copy(w_v.at[b], deg_sh.at[idx_v.at[b]],
                                  ssems[b], add=True)
    for k in range(max(0, _NCH - 2), _NCH):
        sds[k].wait()
    plsc.subcore_barrier()
    pltpu.sync_copy(
        deg_sh.at[pl.ds(sid * _NPT, _NPT)],
        out_hbm.at[cid, pl.ds(sid * _NPT, _NPT)],
    )


_NB = 3


@functools.partial(
    pl.kernel,
    out_type=jax.ShapeDtypeStruct((_NC, _NP, _DH), jnp.float32),
    mesh=_mesh,
    scratch_types=[
        pltpu.VMEM((_NB, _C), jnp.int32),
        pltpu.VMEM((_NB, _C), jnp.int32),
        pltpu.VMEM((_NB, _C), jnp.float32),
        pltpu.VMEM((_NB, _C, _DH), jnp.float32),
        pltpu.VMEM((_NPT, _DH), jnp.float32),
        pltpu.VMEM_SHARED((_NP, _DH), jnp.float32),
        [pltpu.SemaphoreType.DMA] * _NB,
        [pltpu.SemaphoreType.DMA] * _NB,
    ],
    compiler_params=pltpu.CompilerParams(needs_layout_passes=False, use_tc_tiling_on_sc=False),
)
def _sc_agg(ei_hbm, ew_hbm, h_hbm, out_hbm,
            si_v, di_v, w_v, rows_v, zer_v, agg_sh, gsems, ssems):
    cid = lax.axis_index("c")
    sid = lax.axis_index("s")

    @plsc.parallel_loop(0, _NPT, unroll=8)
    def zbody(i):
        zer_v[i] = jnp.zeros((_DH,), jnp.float32)

    pltpu.sync_copy(zer_v, agg_sh.at[pl.ds(sid * _NPT, _NPT)])
    plsc.subcore_barrier()

    base = cid * (_E // _NC) + sid * _EPW

    gds = [None] * _NCH
    sds = [None] * _NCH

    def prefetch(k):
        b = k % _NB
        off = base + k * _C
        pltpu.sync_copy(ei_hbm.at[0, pl.ds(off, _C)], si_v.at[b])
        pltpu.sync_copy(ei_hbm.at[1, pl.ds(off, _C)], di_v.at[b])
        pltpu.sync_copy(ew_hbm.at[pl.ds(off, _C)], w_v.at[b])
        gds[k] = pltpu.async_copy(h_hbm.at[si_v.at[b]], rows_v.at[b],
                                  gsems[b])

    prefetch(0)
    for k in range(_NCH):
        b = k % _NB
        if k - 2 >= 0:
            sds[k - 2].wait()
        if k + 1 < _NCH:
            prefetch(k + 1)
        gds[k].wait()

        wb = w_v.at[b]
        rb = rows_v.at[b]

        @plsc.parallel_loop(0, _C, unroll=16)
        def ebody(j):
            ws = plsc.load_gather(wb, [jnp.full((16,), j, jnp.int32)])
            rb[j] = rb[j] * ws

        sds[k] = pltpu.async_copy(rows_v.at[b], agg_sh.at[di_v.at[b]],
                                  ssems[b], add=True)
    for k in range(max(0, _NCH - 2), _NCH):
        sds[k].wait()
    plsc.subcore_barrier()
    pltpu.sync_copy(
        agg_sh.at[pl.ds(sid * _NPT, _NPT)],
        out_hbm.at[cid, pl.ds(sid * _NPT, _NPT)],
    )


def _tc1_body(x_ref, w1_ref, degt_ref, h1s_ref, dis_ref):
    deg = degt_ref[:, 0:1] + degt_ref[:, 1:2] + 1.0
    dis = jnp.where(deg > 0, lax.rsqrt(jnp.maximum(deg, 1e-12)), 0.0)
    h1 = jnp.dot(x_ref[...], w1_ref[...], preferred_element_type=jnp.float32)
    h1s_ref[...] = h1 * dis
    dis_ref[...] = dis


_tc1 = pl.pallas_call(
    _tc1_body,
    out_shape=(
        jax.ShapeDtypeStruct((_NP, _DH), jnp.float32),
        jax.ShapeDtypeStruct((_NP, 1), jnp.float32),
    ),
)


def _tc2_body(aggp_ref, h1s_ref, dis_ref, b1_ref, w2_ref, zs_ref):
    dis = dis_ref[...]
    a1 = (aggp_ref[0] + aggp_ref[1] + h1s_ref[...]) * dis + b1_ref[...]
    h2 = jnp.maximum(a1, 0.0)
    z = jnp.dot(h2, w2_ref[...], preferred_element_type=jnp.float32)
    zs_ref[...] = z * dis


_tc2 = pl.pallas_call(
    _tc2_body,
    out_shape=jax.ShapeDtypeStruct((_NP, _DH), jnp.float32),
)


def _tc3_body(qp_ref, zs_ref, dis_ref, b2_ref, out_ref):
    dis = dis_ref[...]
    logits = (qp_ref[0] + qp_ref[1] + zs_ref[...]) * dis + b2_ref[...]
    mask = lax.broadcasted_iota(jnp.int32, (1, _DH), 1) < _DOUT
    lm = jnp.where(mask, logits, jnp.float32(-3.4e38))
    m = jnp.max(lm, axis=1, keepdims=True)
    e = jnp.where(mask, jnp.exp(logits - m), 0.0)
    s = jnp.sum(e, axis=1, keepdims=True)
    out_ref[...] = logits - m - jnp.log(s)


_tc3 = pl.pallas_call(
    _tc3_body,
    out_shape=jax.ShapeDtypeStruct((_NP, _DH), jnp.float32),
)


def kernel(x, edge_index, edge_weight, W1, b1, W2, b2):
    xp = jnp.pad(x, ((0, _NP - _N), (0, 0)))
    w2p = jnp.pad(W2, ((0, 0), (0, _DH - _DOUT)))
    b1r = b1.reshape(1, _DH)
    b2r = jnp.pad(b2, (0, _DH - _DOUT)).reshape(1, _DH)

    degp = _sc_deg(edge_index, edge_weight)
    degt = degp.T
    h1s, dis = _tc1(xp, W1, degt)
    aggp = _sc_agg(edge_index, edge_weight, h1s)
    zs = _tc2(aggp, h1s, dis, b1r, w2p)
    qp = _sc_agg(edge_index, edge_weight, zs)
    outp = _tc3(qp, zs, dis, b2r)
    return outp[:_N, :_DOUT]


# speedup vs baseline: 57.1935x; 1.0090x over previous
import functools

import jax
import jax.numpy as jnp
from jax import lax
from jax.experimental import pallas as pl
from jax.experimental.pallas import tpu as pltpu
from jax.experimental.pallas import tpu_sc as plsc

_N = 10000
_E = 320000
_DIN = 128
_DH = 16
_DOUT = 10

_NC = 2
_NS = 16
_NP = 10240
_NPT = _NP // _NS
_EPW = _E // (_NC * _NS)
_C = 1000
_NCH = _EPW // _C

_mesh = plsc.VectorSubcoreMesh(
    core_axis_name="c", subcore_axis_name="s", num_cores=_NC, num_subcores=_NS
)


@functools.partial(
    pl.kernel,
    out_type=jax.ShapeDtypeStruct((_NC, _NP), jnp.float32),
    mesh=_mesh,
    scratch_types=[
        pltpu.VMEM((3, _C), jnp.int32),
        pltpu.VMEM((3, _C), jnp.float32),
        pltpu.VMEM((_NPT,), jnp.float32),
        pltpu.VMEM_SHARED((_NP,), jnp.float32),
        [pltpu.SemaphoreType.DMA] * 3,
    ],
    compiler_params=pltpu.CompilerParams(needs_layout_passes=False, use_tc_tiling_on_sc=False),
)
def _sc_deg(ei_hbm, ew_hbm, out_hbm, idx_v, w_v, zer_v, deg_sh, ssems):
    cid = lax.axis_index("c")
    sid = lax.axis_index("s")

    @plsc.parallel_loop(0, _NPT // 16, unroll=8)
    def zbody(i):
        zer_v[pl.ds(i * 16, 16)] = jnp.zeros((16,), jnp.float32)

    pltpu.sync_copy(zer_v, deg_sh.at[pl.ds(sid * _NPT, _NPT)])
    plsc.subcore_barrier()

    base = cid * (_E // _NC) + sid * _EPW

    def load(k):
        b = k % 3
        off = base + k * _C
        pltpu.sync_copy(ei_hbm.at[1, pl.ds(off, _C)], idx_v.at[b])
        pltpu.sync_copy(ew_hbm.at[pl.ds(off, _C)], w_v.at[b])

    sds = [None] * _NCH
    load(0)
    for k in range(_NCH):
        b = k % 3
        if k - 2 >= 0:
            sds[k - 2].wait()
        if k + 1 < _NCH:
            load(k + 1)
        sds[k] = pltpu.async_copy(w_v.at[b], deg_sh.at[idx_v.at[b]],
                                  ssems[b], add=True)
    for k in range(max(0, _NCH - 2), _NCH):
        sds[k].wait()
    plsc.subcore_barrier()
    pltpu.sync_copy(
        deg_sh.at[pl.ds(sid * _NPT, _NPT)],
        out_hbm.at[cid, pl.ds(sid * _NPT, _NPT)],
    )


_NB = 3


@functools.partial(
    pl.kernel,
    out_type=jax.ShapeDtypeStruct((_NC, _NP, _DH), jnp.float32),
    mesh=_mesh,
    scratch_types=[
        pltpu.VMEM((_NB, _C), jnp.int32),
        pltpu.VMEM((_NB, _C), jnp.int32),
        pltpu.VMEM((_NB, _C), jnp.float32),
        pltpu.VMEM((_NB, _C, _DH), jnp.float32),
        pltpu.VMEM((_NPT, _DH), jnp.float32),
        pltpu.VMEM_SHARED((_NP, _DH), jnp.float32),
        [pltpu.SemaphoreType.DMA] * _NB,
        [pltpu.SemaphoreType.DMA] * _NB,
    ],
    compiler_params=pltpu.CompilerParams(needs_layout_passes=False, use_tc_tiling_on_sc=False),
)
def _sc_agg(ei_hbm, ew_hbm, h_hbm, out_hbm,
            si_v, di_v, w_v, rows_v, zer_v, agg_sh, gsems, ssems):
    cid = lax.axis_index("c")
    sid = lax.axis_index("s")

    @plsc.parallel_loop(0, _NPT, unroll=8)
    def zbody(i):
        zer_v[i] = jnp.zeros((_DH,), jnp.float32)

    pltpu.sync_copy(zer_v, agg_sh.at[pl.ds(sid * _NPT, _NPT)])
    plsc.subcore_barrier()

    base = cid * (_E // _NC) + sid * _EPW

    gds = [None] * _NCH
    sds = [None] * _NCH

    def prefetch(k):
        b = k % _NB
        off = base + k * _C
        pltpu.sync_copy(ei_hbm.at[0, pl.ds(off, _C)], si_v.at[b])
        pltpu.sync_copy(ei_hbm.at[1, pl.ds(off, _C)], di_v.at[b])
        pltpu.sync_copy(ew_hbm.at[pl.ds(off, _C)], w_v.at[b])
        gds[k] = pltpu.async_copy(h_hbm.at[si_v.at[b]], rows_v.at[b],
                                  gsems[b])

    prefetch(0)
    for k in range(_NCH):
        b = k % _NB
        if k - 2 >= 0:
            sds[k - 2].wait()
        if k + 1 < _NCH:
            prefetch(k + 1)
        gds[k].wait()

        wb = w_v.at[b]
        rb = rows_v.at[b]

        @plsc.parallel_loop(0, _C, unroll=16)
        def ebody(j):
            ws = plsc.load_gather(wb, [jnp.full((16,), j, jnp.int32)])
            rb[j] = rb[j] * ws

        sds[k] = pltpu.async_copy(rows_v.at[b], agg_sh.at[di_v.at[b]],
                                  ssems[b], add=True)
    for k in range(max(0, _NCH - 2), _NCH):
        sds[k].wait()
    plsc.subcore_barrier()
    pltpu.sync_copy(
        agg_sh.at[pl.ds(sid * _NPT, _NPT)],
        out_hbm.at[cid, pl.ds(sid * _NPT, _NPT)],
    )


def _tc1_body(x_ref, w1_ref, degt_ref, h1s_ref, dis_ref):
    deg = degt_ref[:, 0:1] + degt_ref[:, 1:2] + 1.0
    dis = jnp.where(deg > 0, lax.rsqrt(jnp.maximum(deg, 1e-12)), 0.0)
    h1 = jnp.dot(x_ref[...], w1_ref[...], preferred_element_type=jnp.float32)
    h1s_ref[...] = h1 * dis
    dis_ref[...] = dis


_tc1 = pl.pallas_call(
    _tc1_body,
    out_shape=(
        jax.ShapeDtypeStruct((_NP, _DH), jnp.float32),
        jax.ShapeDtypeStruct((_NP, 1), jnp.float32),
    ),
)


def _tc2_body(aggp_ref, h1s_ref, dis_ref, b1_ref, w2_ref, zs_ref):
    dis = dis_ref[...]
    a1 = (aggp_ref[0] + aggp_ref[1] + h1s_ref[...]) * dis + b1_ref[...]
    h2 = jnp.maximum(a1, 0.0)
    z = jnp.dot(h2, w2_ref[...], preferred_element_type=jnp.float32)
    zs_ref[...] = z * dis


_tc2 = pl.pallas_call(
    _tc2_body,
    out_shape=jax.ShapeDtypeStruct((_NP, _DH), jnp.float32),
)


def _tc3_body(qp_ref, zs_ref, dis_ref, b2_ref, out_ref):
    qs = qp_ref[0, : _N, :] + qp_ref[1, : _N, :] + zs_ref[: _N, :]
    logits = qs * dis_ref[: _N, :] + b2_ref[...]
    mask = lax.broadcasted_iota(jnp.int32, (1, _DH), 1) < _DOUT
    lm = jnp.where(mask, logits, jnp.float32(-3.4e38))
    m = jnp.max(lm, axis=1, keepdims=True)
    e = jnp.where(mask, jnp.exp(logits - m), 0.0)
    sm = jnp.sum(e, axis=1, keepdims=True)
    out_ref[...] = (logits - m - jnp.log(sm))[:, : _DOUT]


_tc3 = pl.pallas_call(
    _tc3_body,
    out_shape=jax.ShapeDtypeStruct((_N, _DOUT), jnp.float32),
)


def kernel(x, edge_index, edge_weight, W1, b1, W2, b2):
    xp = jnp.pad(x, ((0, _NP - _N), (0, 0)))
    w2p = jnp.pad(W2, ((0, 0), (0, _DH - _DOUT)))
    b1r = b1.reshape(1, _DH)
    b2r = jnp.pad(b2, (0, _DH - _DOUT)).reshape(1, _DH)

    degp = _sc_deg(edge_index, edge_weight)
    h1s, dis = _tc1(xp, W1, degp.T)
    aggp = _sc_agg(edge_index, edge_weight, h1s)
    zs = _tc2(aggp, h1s, dis, b1r, w2p)
    qp = _sc_agg(edge_index, edge_weight, zs)
    return _tc3(qp, zs, dis, b2r)


# speedup vs baseline: 58.0743x; 1.0154x over previous
import functools

import jax
import jax.numpy as jnp
from jax import lax
from jax.experimental import pallas as pl
from jax.experimental.pallas import tpu as pltpu
from jax.experimental.pallas import tpu_sc as plsc

_N = 10000
_E = 320000
_DIN = 128
_DH = 16
_DOUT = 10

_NC = 2
_NS = 16
_NP = 10240
_NPT = _NP // _NS
_EPW = _E // (_NC * _NS)
_C = 1000
_NCH = _EPW // _C

_mesh = plsc.VectorSubcoreMesh(
    core_axis_name="c", subcore_axis_name="s", num_cores=_NC, num_subcores=_NS
)


@functools.partial(
    pl.kernel,
    out_type=jax.ShapeDtypeStruct((_NC, _NP), jnp.float32),
    mesh=_mesh,
    scratch_types=[
        pltpu.VMEM((3, _C), jnp.int32),
        pltpu.VMEM((3, _C), jnp.float32),
        pltpu.VMEM((_NPT,), jnp.float32),
        pltpu.VMEM_SHARED((_NP,), jnp.float32),
        [pltpu.SemaphoreType.DMA] * 3,
    ],
    compiler_params=pltpu.CompilerParams(needs_layout_passes=False, use_tc_tiling_on_sc=False),
)
def _sc_deg(ei_hbm, ew_hbm, out_hbm, idx_v, w_v, zer_v, deg_sh, ssems):
    cid = lax.axis_index("c")
    sid = lax.axis_index("s")

    @plsc.parallel_loop(0, _NPT // 16, unroll=8)
    def zbody(i):
        zer_v[pl.ds(i * 16, 16)] = jnp.zeros((16,), jnp.float32)

    pltpu.sync_copy(zer_v, deg_sh.at[pl.ds(sid * _NPT, _NPT)])
    plsc.subcore_barrier()

    base = cid * (_E // _NC) + sid * _EPW

    def load(k):
        b = k % 3
        off = base + k * _C
        pltpu.sync_copy(ei_hbm.at[1, pl.ds(off, _C)], idx_v.at[b])
        pltpu.sync_copy(ew_hbm.at[pl.ds(off, _C)], w_v.at[b])

    sds = [None] * _NCH
    load(0)
    for k in range(_NCH):
        b = k % 3
        if k - 2 >= 0:
            sds[k - 2].wait()
        if k + 1 < _NCH:
            load(k + 1)
        sds[k] = pltpu.async_copy(w_v.at[b], deg_sh.at[idx_v.at[b]],
                                  ssems[b], add=True)
    for k in range(max(0, _NCH - 2), _NCH):
        sds[k].wait()
    plsc.subcore_barrier()
    pltpu.sync_copy(
        deg_sh.at[pl.ds(sid * _NPT, _NPT)],
        out_hbm.at[cid, pl.ds(sid * _NPT, _NPT)],
    )


_NB = 3


@functools.partial(
    pl.kernel,
    out_type=jax.ShapeDtypeStruct((_NC, _NP, _DH), jnp.float32),
    mesh=_mesh,
    scratch_types=[
        pltpu.VMEM((_NB, _C), jnp.int32),
        pltpu.VMEM((_NB, _C), jnp.int32),
        pltpu.VMEM((_NB, _C), jnp.float32),
        pltpu.VMEM((_NB, _C, _DH), jnp.float32),
        pltpu.VMEM((_NPT, _DH), jnp.float32),
        pltpu.VMEM_SHARED((_NP, _DH), jnp.float32),
        [pltpu.SemaphoreType.DMA] * _NB,
        [pltpu.SemaphoreType.DMA] * _NB,
    ],
    compiler_params=pltpu.CompilerParams(needs_layout_passes=False, use_tc_tiling_on_sc=False),
)
def _sc_agg(ei_hbm, ew_hbm, h_hbm, out_hbm,
            si_v, di_v, w_v, rows_v, zer_v, agg_sh, gsems, ssems):
    cid = lax.axis_index("c")
    sid = lax.axis_index("s")

    @plsc.parallel_loop(0, _NPT, unroll=8)
    def zbody(i):
        zer_v[i] = jnp.zeros((_DH,), jnp.float32)

    pltpu.sync_copy(zer_v, agg_sh.at[pl.ds(sid * _NPT, _NPT)])
    plsc.subcore_barrier()

    base = cid * (_E // _NC) + sid * _EPW

    gds = [None] * _NCH
    sds = [None] * _NCH

    def prefetch(k):
        b = k % _NB
        off = base + k * _C
        pltpu.sync_copy(ei_hbm.at[0, pl.ds(off, _C)], si_v.at[b])
        pltpu.sync_copy(ei_hbm.at[1, pl.ds(off, _C)], di_v.at[b])
        pltpu.sync_copy(ew_hbm.at[pl.ds(off, _C)], w_v.at[b])
        gds[k] = pltpu.async_copy(h_hbm.at[si_v.at[b]], rows_v.at[b],
                                  gsems[b])

    prefetch(0)
    for k in range(_NCH):
        b = k % _NB
        if k - 2 >= 0:
            sds[k - 2].wait()
        if k + 1 < _NCH:
            prefetch(k + 1)
        gds[k].wait()

        wb = w_v.at[b]
        rb = rows_v.at[b]

        @plsc.parallel_loop(0, _C, unroll=8)
        def ebody(j):
            ws = plsc.load_gather(wb, [jnp.full((16,), j, jnp.int32)])
            rb[j] = rb[j] * ws

        sds[k] = pltpu.async_copy(rows_v.at[b], agg_sh.at[di_v.at[b]],
                                  ssems[b], add=True)
    for k in range(max(0, _NCH - 2), _NCH):
        sds[k].wait()
    plsc.subcore_barrier()
    pltpu.sync_copy(
        agg_sh.at[pl.ds(sid * _NPT, _NPT)],
        out_hbm.at[cid, pl.ds(sid * _NPT, _NPT)],
    )


def _tc1_body(x_ref, w1_ref, degt_ref, h1s_ref, dis_ref):
    deg = degt_ref[:, 0:1] + degt_ref[:, 1:2] + 1.0
    dis = jnp.where(deg > 0, lax.rsqrt(jnp.maximum(deg, 1e-12)), 0.0)
    h1 = jnp.dot(x_ref[...], w1_ref[...], preferred_element_type=jnp.float32)
    h1s_ref[...] = h1 * dis
    dis_ref[...] = dis


_tc1 = pl.pallas_call(
    _tc1_body,
    out_shape=(
        jax.ShapeDtypeStruct((_NP, _DH), jnp.float32),
        jax.ShapeDtypeStruct((_NP, 1), jnp.float32),
    ),
)


def _tc2_body(aggp_ref, h1s_ref, dis_ref, b1_ref, w2_ref, zs_ref):
    dis = dis_ref[...]
    a1 = (aggp_ref[0] + aggp_ref[1] + h1s_ref[...]) * dis + b1_ref[...]
    h2 = jnp.maximum(a1, 0.0)
    z = jnp.dot(h2, w2_ref[...], preferred_element_type=jnp.float32)
    zs_ref[...] = z * dis


_tc2 = pl.pallas_call(
    _tc2_body,
    out_shape=jax.ShapeDtypeStruct((_NP, _DH), jnp.float32),
)


def _tc3_body(qp_ref, zs_ref, dis_ref, b2_ref, out_ref):
    qs = qp_ref[0, : _N, :] + qp_ref[1, : _N, :] + zs_ref[: _N, :]
    logits = qs * dis_ref[: _N, :] + b2_ref[...]
    mask = lax.broadcasted_iota(jnp.int32, (1, _DH), 1) < _DOUT
    lm = jnp.where(mask, logits, jnp.float32(-3.4e38))
    m = jnp.max(lm, axis=1, keepdims=True)
    e = jnp.where(mask, jnp.exp(logits - m), 0.0)
    sm = jnp.sum(e, axis=1, keepdims=True)
    out_ref[...] = (logits - m - jnp.log(sm))[:, : _DOUT]


_tc3 = pl.pallas_call(
    _tc3_body,
    out_shape=jax.ShapeDtypeStruct((_N, _DOUT), jnp.float32),
)


def kernel(x, edge_index, edge_weight, W1, b1, W2, b2):
    xp = jnp.pad(x, ((0, _NP - _N), (0, 0)))
    w2p = jnp.pad(W2, ((0, 0), (0, _DH - _DOUT)))
    b1r = b1.reshape(1, _DH)
    b2r = jnp.pad(b2, (0, _DH - _DOUT)).reshape(1, _DH)

    degp = _sc_deg(edge_index, edge_weight)
    h1s, dis = _tc1(xp, W1, degp.T)
    aggp = _sc_agg(edge_index, edge_weight, h1s)
    zs = _tc2(aggp, h1s, dis, b1r, w2p)
    qp = _sc_agg(edge_index, edge_weight, zs)
    return _tc3(qp, zs, dis, b2r)


# speedup vs baseline: 60.6753x; 1.0448x over previous
import functools

import jax
import jax.numpy as jnp
from jax import lax
from jax.experimental import pallas as pl
from jax.experimental.pallas import tpu as pltpu
from jax.experimental.pallas import tpu_sc as plsc

_N = 10000
_E = 320000
_DIN = 128
_DH = 16
_DOUT = 10

_NC = 2
_NS = 16
_NP = 10240
_NPT = _NP // _NS
_EPW = _E // (_NC * _NS)
_C = 1000
_NCH = _EPW // _C

_mesh = plsc.VectorSubcoreMesh(
    core_axis_name="c", subcore_axis_name="s", num_cores=_NC, num_subcores=_NS
)


@functools.partial(
    pl.kernel,
    out_type=jax.ShapeDtypeStruct((_NC, _NP), jnp.float32),
    mesh=_mesh,
    scratch_types=[
        pltpu.VMEM((3, _C), jnp.int32),
        pltpu.VMEM((3, _C), jnp.float32),
        pltpu.VMEM((_NPT,), jnp.float32),
        pltpu.VMEM_SHARED((_NP,), jnp.float32),
        [pltpu.SemaphoreType.DMA] * 3,
    ],
    compiler_params=pltpu.CompilerParams(needs_layout_passes=False, use_tc_tiling_on_sc=False),
)
def _sc_deg(ei_hbm, ew_hbm, out_hbm, idx_v, w_v, zer_v, deg_sh, ssems):
    cid = lax.axis_index("c")
    sid = lax.axis_index("s")

    @plsc.parallel_loop(0, _NPT // 16, unroll=8)
    def zbody(i):
        zer_v[pl.ds(i * 16, 16)] = jnp.zeros((16,), jnp.float32)

    pltpu.sync_copy(zer_v, deg_sh.at[pl.ds(sid * _NPT, _NPT)])
    plsc.subcore_barrier()

    base = cid * (_E // _NC) + sid * _EPW

    def load(k):
        b = k % 3
        off = base + k * _C
        pltpu.sync_copy(ei_hbm.at[1, pl.ds(off, _C)], idx_v.at[b])
        pltpu.sync_copy(ew_hbm.at[pl.ds(off, _C)], w_v.at[b])

    sds = [None] * _NCH
    load(0)
    for k in range(_NCH):
        b = k % 3
        if k - 2 >= 0:
            sds[k - 2].wait()
        if k + 1 < _NCH:
            load(k + 1)
        sds[k] = pltpu.async_copy(w_v.at[b], deg_sh.at[idx_v.at[b]],
                                  ssems[b], add=True)
    for k in range(max(0, _NCH - 2), _NCH):
        sds[k].wait()
    plsc.subcore_barrier()
    pltpu.sync_copy(
        deg_sh.at[pl.ds(sid * _NPT, _NPT)],
        out_hbm.at[cid, pl.ds(sid * _NPT, _NPT)],
    )


_NB = 3


@functools.partial(
    pl.kernel,
    out_type=jax.ShapeDtypeStruct((_NC, _NP, _DH), jnp.float32),
    mesh=_mesh,
    scratch_types=[
        pltpu.VMEM((_NB, _C), jnp.int32),
        pltpu.VMEM((_NB, _C), jnp.int32),
        pltpu.VMEM((_NB, _C), jnp.float32),
        pltpu.VMEM((_NB, _C, _DH), jnp.float32),
        pltpu.VMEM((_NPT, _DH), jnp.float32),
        pltpu.VMEM_SHARED((_NP, _DH), jnp.float32),
        [pltpu.SemaphoreType.DMA] * _NB,
        [pltpu.SemaphoreType.DMA] * _NB,
    ],
    compiler_params=pltpu.CompilerParams(needs_layout_passes=False, use_tc_tiling_on_sc=False),
)
def _sc_agg(ei_hbm, ew_hbm, h_hbm, out_hbm,
            si_v, di_v, w_v, rows_v, zer_v, agg_sh, gsems, ssems):
    cid = lax.axis_index("c")
    sid = lax.axis_index("s")

    @plsc.parallel_loop(0, _NPT, unroll=8)
    def zbody(i):
        zer_v[i] = jnp.zeros((_DH,), jnp.float32)

    pltpu.sync_copy(zer_v, agg_sh.at[pl.ds(sid * _NPT, _NPT)])
    plsc.subcore_barrier()

    base = cid * (_E // _NC) + sid * _EPW

    gds = [None] * _NCH
    sds = [None] * _NCH

    def prefetch(k):
        b = k % _NB
        off = base + k * _C
        pltpu.sync_copy(ei_hbm.at[0, pl.ds(off, _C)], si_v.at[b])
        pltpu.sync_copy(ei_hbm.at[1, pl.ds(off, _C)], di_v.at[b])
        pltpu.sync_copy(ew_hbm.at[pl.ds(off, _C)], w_v.at[b])
        gds[k] = pltpu.async_copy(h_hbm.at[si_v.at[b]], rows_v.at[b],
                                  gsems[b])

    prefetch(0)
    for k in range(_NCH):
        b = k % _NB
        if k - 2 >= 0:
            sds[k - 2].wait()
        if k + 1 < _NCH:
            prefetch(k + 1)
        gds[k].wait()

        wb = w_v.at[b]
        rb = rows_v.at[b]

        @plsc.parallel_loop(0, _C // 16, unroll=2)
        def ebody(g):
            w16 = wb[pl.ds(g * 16, 16)]
            for jj in range(16):
                ws = jnp.full((16,), w16[jj])
                rb[g * 16 + jj] = rb[g * 16 + jj] * ws
        del ebody
        wt = wb[pl.ds(_C - 16, 16)]
        for jj in range(16 - (_C - (_C // 16) * 16), 16):
            ws = jnp.full((16,), wt[jj])
            rb[_C - 16 + jj] = rb[_C - 16 + jj] * ws

        sds[k] = pltpu.async_copy(rows_v.at[b], agg_sh.at[di_v.at[b]],
                                  ssems[b], add=True)
    for k in range(max(0, _NCH - 2), _NCH):
        sds[k].wait()
    plsc.subcore_barrier()
    pltpu.sync_copy(
        agg_sh.at[pl.ds(sid * _NPT, _NPT)],
        out_hbm.at[cid, pl.ds(sid * _NPT, _NPT)],
    )


def _tc1_body(x_ref, w1_ref, degt_ref, h1s_ref, dis_ref):
    deg = degt_ref[:, 0:1] + degt_ref[:, 1:2] + 1.0
    dis = jnp.where(deg > 0, lax.rsqrt(jnp.maximum(deg, 1e-12)), 0.0)
    h1 = jnp.dot(x_ref[...], w1_ref[...], preferred_element_type=jnp.float32)
    h1s_ref[...] = h1 * dis
    dis_ref[...] = dis


_tc1 = pl.pallas_call(
    _tc1_body,
    out_shape=(
        jax.ShapeDtypeStruct((_NP, _DH), jnp.float32),
        jax.ShapeDtypeStruct((_NP, 1), jnp.float32),
    ),
)


def _tc2_body(aggp_ref, h1s_ref, dis_ref, b1_ref, w2_ref, zs_ref):
    dis = dis_ref[...]
    a1 = (aggp_ref[0] + aggp_ref[1] + h1s_ref[...]) * dis + b1_ref[...]
    h2 = jnp.maximum(a1, 0.0)
    z = jnp.dot(h2, w2_ref[...], preferred_element_type=jnp.float32)
    zs_ref[...] = z * dis


_tc2 = pl.pallas_call(
    _tc2_body,
    out_shape=jax.ShapeDtypeStruct((_NP, _DH), jnp.float32),
)


def _tc3_body(qp_ref, zs_ref, dis_ref, b2_ref, out_ref):
    qs = qp_ref[0, : _N, :] + qp_ref[1, : _N, :] + zs_ref[: _N, :]
    logits = qs * dis_ref[: _N, :] + b2_ref[...]
    mask = lax.broadcasted_iota(jnp.int32, (1, _DH), 1) < _DOUT
    lm = jnp.where(mask, logits, jnp.float32(-3.4e38))
    m = jnp.max(lm, axis=1, keepdims=True)
    e = jnp.where(mask, jnp.exp(logits - m), 0.0)
    sm = jnp.sum(e, axis=1, keepdims=True)
    out_ref[...] = (logits - m - jnp.log(sm))[:, : _DOUT]


_tc3 = pl.pallas_call(
    _tc3_body,
    out_shape=jax.ShapeDtypeStruct((_N, _DOUT), jnp.float32),
)


def kernel(x, edge_index, edge_weight, W1, b1, W2, b2):
    xp = jnp.pad(x, ((0, _NP - _N), (0, 0)))
    w2p = jnp.pad(W2, ((0, 0), (0, _DH - _DOUT)))
    b1r = b1.reshape(1, _DH)
    b2r = jnp.pad(b2, (0, _DH - _DOUT)).reshape(1, _DH)

    degp = _sc_deg(edge_index, edge_weight)
    h1s, dis = _tc1(xp, W1, degp.T)
    aggp = _sc_agg(edge_index, edge_weight, h1s)
    zs = _tc2(aggp, h1s, dis, b1r, w2p)
    qp = _sc_agg(edge_index, edge_weight, zs)
    return _tc3(qp, zs, dis, b2r)
